# Initial kernel scaffold; baseline (speedup 1.0000x reference)
import jax
import jax.numpy as jnp
from jax.experimental import pallas as pl


def kernel(x, edge_index, W1, b1, g1, be1, W2, b2, g2, be2, W3, b3, g3, be3, Wl, bl):
    raise NotImplementedError("write your pallas kernel here")


# speedup vs baseline: 22.3028x; 22.3028x over previous
import functools

import jax
import jax.numpy as jnp
from jax import lax
from jax.experimental import pallas as pl
from jax.experimental.pallas import tpu as pltpu
from jax.experimental.pallas import tpu_sc as plsc

N = 10000
E = 320000
D = 128
EPS = 1e-5

NC = 2
NS = 16
NW = NC * NS
EPT = E // NW
CH = 125
NCH = EPT // CH
IB = 16
NB = NCH // IB
NP = 10240
RPS = NP // NS
RC = 128
RCH = RPS // RC

R = 1000
GRID = N // R

_mesh = plsc.VectorSubcoreMesh(core_axis_name="c", subcore_axis_name="s")


@functools.partial(
    pl.kernel,
    out_type=jax.ShapeDtypeStruct((NC, NP, D), jnp.float32),
    mesh=_mesh,
    scratch_types=[
        pltpu.VMEM((2, 2, IB, CH), jnp.int32),
        pltpu.VMEM((CH, D), jnp.float32),
        pltpu.VMEM((RC, D), jnp.float32),
        pltpu.VMEM_SHARED((NP, D), jnp.float32),
        pltpu.SemaphoreType.DMA((2,)),
    ],
)
def _deg_kernel(e_hbm, out_hbm, idx_v, ones_v, zeros_v, acc, isems):
    c = lax.axis_index("c")
    s = lax.axis_index("s")
    wid = c * NS + s

    one = jnp.full((16,), 1.0, jnp.float32)
    zero = jnp.zeros((16,), jnp.float32)

    def _fill(i, carry):
        for k in range(D // 16):
            ones_v[i, pl.ds(k * 16, 16)] = one
        return carry

    lax.fori_loop(0, CH, _fill, 0)

    def _fillz(i, carry):
        for k in range(D // 16):
            zeros_v[i, pl.ds(k * 16, 16)] = zero
        return carry

    lax.fori_loop(0, RC, _fillz, 0)

    for k in range(RCH):
        pltpu.sync_copy(zeros_v, acc.at[pl.ds(s * RPS + k * RC, RC)])
    plsc.subcore_barrier()

    pltpu.sync_copy(e_hbm.at[wid, 0], idx_v.at[0])

    def _block(j, carry):
        jb = lax.rem(j, 2)

        @pl.when(j > 0)
        def _():
            pltpu.make_async_copy(e_hbm.at[wid, j], idx_v.at[jb],
                                  isems.at[jb]).wait()

        @pl.when(j + 1 < NB)
        def _():
            nb = lax.rem(j + 1, 2)
            pltpu.async_copy(e_hbm.at[wid, j + 1], idx_v.at[nb], isems.at[nb])

        for i in range(IB):
            pltpu.sync_copy(ones_v, acc.at[idx_v.at[jb, 1, i]], add=True)
        return carry

    lax.fori_loop(0, NB, _block, 0)
    plsc.subcore_barrier()

    for k in range(RCH):
        r0 = s * RPS + k * RC
        pltpu.sync_copy(acc.at[pl.ds(r0, RC)], out_hbm.at[c, pl.ds(r0, RC)])


@functools.partial(
    pl.kernel,
    out_type=jax.ShapeDtypeStruct((NC, NP, D), jnp.float32),
    mesh=_mesh,
    scratch_types=[
        pltpu.VMEM((2, 2, IB, CH), jnp.int32),
        pltpu.VMEM((2, RC, D), jnp.float32),
        pltpu.VMEM_SHARED((NP, D), jnp.float32),
        pltpu.SemaphoreType.DMA((2,)),
        pltpu.SemaphoreType.DMA((2,)),
    ],
)
def _agg_kernel(u_hbm, e_hbm, out_hbm, idx_v, rows_v, acc, sems, isems):
    c = lax.axis_index("c")
    s = lax.axis_index("s")
    wid = c * NS + s

    zero = jnp.zeros((16,), jnp.float32)

    def _fill(i, carry):
        for k in range(D // 16):
            rows_v[0, i, pl.ds(k * 16, 16)] = zero
        return carry

    lax.fori_loop(0, RC, _fill, 0)

    for k in range(RCH):
        pltpu.sync_copy(rows_v.at[0], acc.at[pl.ds(s * RPS + k * RC, RC)])
    plsc.subcore_barrier()

    pltpu.sync_copy(e_hbm.at[wid, 0], idx_v.at[0])

    def _block(j, carry):
        jb = lax.rem(j, 2)

        @pl.when(j > 0)
        def _():
            pltpu.make_async_copy(e_hbm.at[wid, j], idx_v.at[jb],
                                  isems.at[jb]).wait()

        @pl.when(j + 1 < NB)
        def _():
            nb = lax.rem(j + 1, 2)
            pltpu.async_copy(e_hbm.at[wid, j + 1], idx_v.at[nb], isems.at[nb])

        pltpu.async_copy(u_hbm.at[idx_v.at[jb, 0, 0]],
                         rows_v.at[0, pl.ds(0, CH)], sems.at[0])
        for i in range(IB):
            sl = i % 2
            if i + 1 < IB:
                nsl = (i + 1) % 2
                pltpu.async_copy(u_hbm.at[idx_v.at[jb, 0, i + 1]],
                                 rows_v.at[nsl, pl.ds(0, CH)], sems.at[nsl])
            pltpu.make_async_copy(u_hbm.at[idx_v.at[jb, 0, i]],
                                  rows_v.at[sl, pl.ds(0, CH)],
                                  sems.at[sl]).wait()
            pltpu.sync_copy(rows_v.at[sl, pl.ds(0, CH)],
                            acc.at[idx_v.at[jb, 1, i]], add=True)
        return carry

    lax.fori_loop(0, NB, _block, 0)
    plsc.subcore_barrier()

    for k in range(RCH):
        r0 = s * RPS + k * RC
        pltpu.sync_copy(acc.at[pl.ds(r0, RC)], out_hbm.at[c, pl.ds(r0, RC)])


def _dinv(d0, d1):
    return lax.rsqrt(d0 + d1 + 1.0)


def _k1_body(x_ref, w_ref, d0_ref, d1_ref, u_ref):
    u_ref[...] = jnp.dot(x_ref[...], w_ref[...],
                         preferred_element_type=jnp.float32) * _dinv(
                             d0_ref[...], d1_ref[...])


_k1 = pl.pallas_call(
    _k1_body,
    grid=(GRID,),
    in_specs=[
        pl.BlockSpec((R, D), lambda i: (i, 0)),
        pl.BlockSpec((D, D), lambda i: (0, 0)),
        pl.BlockSpec((R, 1), lambda i: (i, 0)),
        pl.BlockSpec((R, 1), lambda i: (i, 0)),
    ],
    out_specs=pl.BlockSpec((R, D), lambda i: (i, 0)),
    out_shape=jax.ShapeDtypeStruct((N, D), jnp.float32),
)


def _k2_body(t0_ref, t1_ref, u_ref, d0_ref, d1_ref, b_ref,
             y_ref, s_ref, q_ref):
    dinv = _dinv(d0_ref[...], d1_ref[...])
    y = (t0_ref[...] + t1_ref[...] + u_ref[...]) * dinv + b_ref[...]
    y_ref[...] = y

    @pl.when(pl.program_id(0) == 0)
    def _():
        s_ref[...] = jnp.zeros_like(s_ref)
        q_ref[...] = jnp.zeros_like(q_ref)

    s_ref[...] += jnp.sum(y, axis=0, keepdims=True)
    q_ref[...] += jnp.sum(y * y, axis=0, keepdims=True)


_k2 = pl.pallas_call(
    _k2_body,
    grid=(GRID,),
    in_specs=[
        pl.BlockSpec((R, D), lambda i: (i, 0)),
        pl.BlockSpec((R, D), lambda i: (i, 0)),
        pl.BlockSpec((R, D), lambda i: (i, 0)),
        pl.BlockSpec((R, 1), lambda i: (i, 0)),
        pl.BlockSpec((R, 1), lambda i: (i, 0)),
        pl.BlockSpec((1, D), lambda i: (0, 0)),
    ],
    out_specs=[
        pl.BlockSpec((R, D), lambda i: (i, 0)),
        pl.BlockSpec((1, D), lambda i: (0, 0)),
        pl.BlockSpec((1, D), lambda i: (0, 0)),
    ],
    out_shape=[
        jax.ShapeDtypeStruct((N, D), jnp.float32),
        jax.ShapeDtypeStruct((1, D), jnp.float32),
        jax.ShapeDtypeStruct((1, D), jnp.float32),
    ],
)


def _bn_relu(y, s, q, g, be):
    m = s * (1.0 / N)
    var = q * (1.0 / N) - m * m
    scale = g * lax.rsqrt(var + EPS)
    return jnp.maximum((y - m) * scale + be, 0.0)


def _k3_body(y_ref, s_ref, q_ref, g_ref, be_ref, w_ref, d0_ref, d1_ref,
             u_ref):
    z = _bn_relu(y_ref[...], s_ref[...], q_ref[...], g_ref[...], be_ref[...])
    u_ref[...] = jnp.dot(z, w_ref[...],
                         preferred_element_type=jnp.float32) * _dinv(
                             d0_ref[...], d1_ref[...])


_k3 = pl.pallas_call(
    _k3_body,
    grid=(GRID,),
    in_specs=[
        pl.BlockSpec((R, D), lambda i: (i, 0)),
        pl.BlockSpec((1, D), lambda i: (0, 0)),
        pl.BlockSpec((1, D), lambda i: (0, 0)),
        pl.BlockSpec((1, D), lambda i: (0, 0)),
        pl.BlockSpec((1, D), lambda i: (0, 0)),
        pl.BlockSpec((D, D), lambda i: (0, 0)),
        pl.BlockSpec((R, 1), lambda i: (i, 0)),
        pl.BlockSpec((R, 1), lambda i: (i, 0)),
    ],
    out_specs=pl.BlockSpec((R, D), lambda i: (i, 0)),
    out_shape=jax.ShapeDtypeStruct((N, D), jnp.float32),
)


def _k4_body(y_ref, s_ref, q_ref, g_ref, be_ref, wl_ref, bl_ref, o_ref):
    z = _bn_relu(y_ref[...], s_ref[...], q_ref[...], g_ref[...], be_ref[...])
    o_ref[...] = jnp.dot(z, wl_ref[...],
                         preferred_element_type=jnp.float32) + bl_ref[...]


_k4 = pl.pallas_call(
    _k4_body,
    grid=(GRID,),
    in_specs=[
        pl.BlockSpec((R, D), lambda i: (i, 0)),
        pl.BlockSpec((1, D), lambda i: (0, 0)),
        pl.BlockSpec((1, D), lambda i: (0, 0)),
        pl.BlockSpec((1, D), lambda i: (0, 0)),
        pl.BlockSpec((1, D), lambda i: (0, 0)),
        pl.BlockSpec((D, 1), lambda i: (0, 0)),
        pl.BlockSpec((1, 1), lambda i: (0, 0)),
    ],
    out_specs=pl.BlockSpec((R, 1), lambda i: (i, 0)),
    out_shape=jax.ShapeDtypeStruct((N, 1), jnp.float32),
)


def kernel(x, edge_index, W1, b1, g1, be1, W2, b2, g2, be2, W3, b3, g3, be3,
           Wl, bl):
    src = edge_index[0].astype(jnp.int32).reshape(NW, NCH, CH)
    dst = edge_index[1].astype(jnp.int32).reshape(NW, NCH, CH)
    e = jnp.stack([src.reshape(NW, NB, IB, CH), dst.reshape(NW, NB, IB, CH)],
                  axis=2)

    degp = _deg_kernel(e)
    d0 = degp[0, :N, 0:1]
    d1 = degp[1, :N, 0:1]

    b1r, g1r, be1r = b1.reshape(1, D), g1.reshape(1, D), be1.reshape(1, D)
    b2r, g2r, be2r = b2.reshape(1, D), g2.reshape(1, D), be2.reshape(1, D)
    b3r, g3r, be3r = b3.reshape(1, D), g3.reshape(1, D), be3.reshape(1, D)

    u1 = _k1(x, W1, d0, d1)
    t1 = _agg_kernel(u1, e)
    y1, s1, q1 = _k2(t1[0, :N], t1[1, :N], u1, d0, d1, b1r)
    u2 = _k3(y1, s1, q1, g1r, be1r, W2, d0, d1)
    t2 = _agg_kernel(u2, e)
    y2, s2, q2 = _k2(t2[0, :N], t2[1, :N], u2, d0, d1, b2r)
    u3 = _k3(y2, s2, q2, g2r, be2r, W3, d0, d1)
    t3 = _agg_kernel(u3, e)
    y3, s3, q3 = _k2(t3[0, :N], t3[1, :N], u3, d0, d1, b3r)
    o = _k4(y3, s3, q3, g3r, be3r, Wl, bl.reshape(1, 1))
    return o.reshape(-1)
